# baseline (speedup 1.0000x reference)
import functools

import jax
import jax.numpy as jnp
from jax import lax
from jax.experimental import pallas as pl
from jax.experimental.pallas import tpu as pltpu
from jax.experimental.pallas import tpu_sc as plsc

IN_DIM = 1000000
DEPTH = 64
SEQ = 200
BATCH = 4096
NC = 2
NS = 16
LANES = 16
NW = NC * NS
SLAB = BATCH // NW
NJ = DEPTH // LANES


def _pos_encoding():
    half = DEPTH // 2
    positions = jnp.arange(SEQ, dtype=jnp.float32)[:, None]
    depths = jnp.arange(half, dtype=jnp.float32)[None, :] / half
    angle_rates = 1.0 / 10000.0 ** depths
    angle_rads = positions * angle_rates
    return jnp.concatenate([jnp.sin(angle_rads), jnp.cos(angle_rads)], axis=-1)


def _make_sc_kernel():
    mesh = plsc.VectorSubcoreMesh(core_axis_name="c", subcore_axis_name="s")

    @functools.partial(
        pl.kernel,
        mesh=mesh,
        compiler_params=pltpu.CompilerParams(needs_layout_passes=False),
        out_type=jax.ShapeDtypeStruct((SEQ, DEPTH, BATCH), jnp.float32),
        scratch_types=[
            pltpu.VMEM((SEQ, SLAB), jnp.int32),
            pltpu.VMEM((SEQ, SLAB), jnp.int32),
            pltpu.VMEM((2, SLAB, 128), jnp.float32),
            pltpu.VMEM((2, DEPTH, SLAB), jnp.float32),
            pltpu.VMEM((SEQ // 2, 2 * DEPTH), jnp.float32),
            pltpu.SemaphoreType.DMA,
            pltpu.SemaphoreType.DMA,
            pltpu.SemaphoreType.DMA,
            pltpu.SemaphoreType.DMA,
        ],
    )
    def k(seqT_hbm, table_hbm, pos_hbm, outT_hbm, idxr_v, idx2_v, rows2_v,
          out_v, pos_v, gsem0, gsem1, wsem0, wsem1):
        wid = lax.axis_index("s") * NC + lax.axis_index("c")
        b0 = wid * SLAB
        iota16 = lax.iota(jnp.int32, LANES)
        dvecs = [iota16 + (LANES * j) for j in range(NJ)]
        gsems = (gsem0, gsem1)
        wsems = (wsem0, wsem1)

        pltpu.sync_copy(pos_hbm, pos_v)
        pltpu.sync_copy(seqT_hbm.at[:, pl.ds(b0, SLAB)], idxr_v)

        def shift_body(s, carry):
            for m in range(SLAB // LANES):
                sl = pl.ds(m * LANES, LANES)
                idx2_v[s, sl] = lax.shift_right_logical(idxr_v[s, sl], 1)
            return carry

        lax.fori_loop(0, SEQ, shift_body, 0)

        def fire_gather(s, buf):
            pltpu.async_copy(
                table_hbm.at[idx2_v.at[s]], rows2_v.at[buf], gsems[buf]
            )

        def wait_gather(buf):
            pltpu.make_async_copy(
                table_hbm.at[idx2_v.at[0]], rows2_v.at[buf], gsems[buf]
            ).wait()

        def fire_write(s, buf):
            pltpu.async_copy(
                out_v.at[buf], outT_hbm.at[s, :, pl.ds(b0, SLAB)], wsems[buf]
            )

        def wait_write(s, buf):
            pltpu.make_async_copy(
                out_v.at[buf], outT_hbm.at[s, :, pl.ds(b0, SLAB)], wsems[buf]
            ).wait()

        fire_gather(0, 0)
        fire_gather(1, 1)

        def iter_body(i, carry):
            for buf in range(2):
                s = 2 * i + buf
                wait_gather(buf)

                @pl.when(i > 0)
                def _():
                    wait_write(s, buf)

                poff = buf * DEPTH
                rows2b = rows2_v.at[buf]
                outb = out_v.at[buf]
                iv16 = jnp.full((LANES,), i, jnp.int32)

                def grp_body(m, gcarry, s=s, poff=poff, rows2b=rows2b,
                             outb=outb, iv16=iv16):
                    o0 = m * LANES
                    off16 = lax.bitwise_and(
                        idxr_v[s, pl.ds(o0, LANES)], 1) * DEPTH
                    row16 = o0 + iota16
                    for d in range(DEPTH):
                        v = plsc.load_gather(rows2b, [row16, off16 + d])
                        p = plsc.load_gather(
                            pos_v, [iv16, jnp.full((LANES,), poff + d,
                                                   jnp.int32)])
                        outb[d, pl.ds(o0, LANES)] = v + p
                    return gcarry

                lax.fori_loop(0, SLAB // LANES, grp_body, 0)

                fire_write(s, buf)

                @pl.when(i < SEQ // 2 - 1)
                def _():
                    fire_gather(s + 2, buf)
            return carry

        lax.fori_loop(0, SEQ // 2, iter_body, 0)
        wait_write(SEQ - 2, 0)
        wait_write(SEQ - 1, 1)

    return k


def kernel(seq, table):
    seq_t = seq.astype(jnp.int32).T
    table2 = table.reshape(IN_DIM // 2, 128)
    pos2 = _pos_encoding().reshape(SEQ // 2, 2 * DEPTH)
    out_t = _make_sc_kernel()(seq_t, table2, pos2)
    return out_t.transpose(2, 0, 1)


# speedup vs baseline: 1.5374x; 1.5374x over previous
import functools

import jax
import jax.numpy as jnp
from jax import lax
from jax.experimental import pallas as pl
from jax.experimental.pallas import tpu as pltpu
from jax.experimental.pallas import tpu_sc as plsc

IN_DIM = 1000000
DEPTH = 64
SEQ = 200
BATCH = 4096
ROWS = BATCH * SEQ
NC = 2
NS = 16
LANES = 16
NW = NC * NS
PER_W = ROWS // NW
CHUNK = 400
NCHUNK = PER_W // CHUNK
GSZ = 100
NG = CHUNK // GSZ
TOTAL_CHUNKS = ROWS // CHUNK


def _pos_encoding():
    half = DEPTH // 2
    positions = jnp.arange(SEQ, dtype=jnp.float32)[:, None]
    depths = jnp.arange(half, dtype=jnp.float32)[None, :] / half
    angle_rates = 1.0 / 10000.0 ** depths
    angle_rads = positions * angle_rates
    return jnp.concatenate([jnp.sin(angle_rads), jnp.cos(angle_rads)], axis=-1)


def _make_sc_kernel():
    mesh = plsc.VectorSubcoreMesh(core_axis_name="c", subcore_axis_name="s")

    @functools.partial(
        pl.kernel,
        mesh=mesh,
        compiler_params=pltpu.CompilerParams(use_tc_tiling_on_sc=False),
        out_type=jax.ShapeDtypeStruct((BATCH, SEQ, DEPTH), jnp.float32),
        scratch_types=[
            pltpu.VMEM((2, NG, GSZ), jnp.int32),
            pltpu.VMEM((2, CHUNK, DEPTH), jnp.float32),
            pltpu.VMEM((CHUNK, DEPTH), jnp.float32),
            pltpu.SemaphoreType.DMA,
            pltpu.SemaphoreType.DMA,
            pltpu.SemaphoreType.DMA,
            pltpu.SemaphoreType.DMA,
        ],
    )
    def k(idx_hbm, table_hbm, pos_hbm, out_hbm, idx_v, rows_v, pos_v,
          gsem0, gsem1, wsem0, wsem1):
        wid = lax.axis_index("s") * NC + lax.axis_index("c")
        c0 = wid * NCHUNK
        gsems = (gsem0, gsem1)
        wsems = (wsem0, wsem1)

        pltpu.sync_copy(pos_hbm, pos_v)

        def fire_chunk(c, buf):
            pltpu.sync_copy(idx_hbm.at[c0 + c], idx_v.at[buf])
            for g in range(NG):
                pltpu.async_copy(
                    table_hbm.at[idx_v.at[buf, g]],
                    rows_v.at[buf, pl.ds(g * GSZ, GSZ)],
                    gsems[buf],
                )

        def wait_gathers(buf):
            for g in range(NG):
                pltpu.make_async_copy(
                    table_hbm.at[idx_v.at[buf, 0]],
                    rows_v.at[buf, pl.ds(g * GSZ, GSZ)],
                    gsems[buf],
                ).wait()

        def fire_writes(c, buf):
            for h in range(2):
                pltpu.async_copy(
                    rows_v.at[buf, pl.ds(h * SEQ, SEQ)],
                    out_hbm.at[2 * (c0 + c) + h],
                    wsems[buf],
                )

        def wait_writes(c, buf):
            for h in range(2):
                pltpu.make_async_copy(
                    rows_v.at[buf, pl.ds(h * SEQ, SEQ)],
                    out_hbm.at[2 * (c0 + c) + h],
                    wsems[buf],
                ).wait()

        fire_chunk(0, 0)

        def iter_body(i, carry):
            for buf in range(2):
                c = 2 * i + buf

                @pl.when(jnp.logical_and(c + 1 < NCHUNK, c >= 1))
                def _():
                    wait_writes(c - 1, 1 - buf)

                @pl.when(c + 1 < NCHUNK)
                def _():
                    fire_chunk(c + 1, 1 - buf)

                wait_gathers(buf)

                def row_body(r, rcarry, buf=buf):
                    for j in range(DEPTH // LANES):
                        sl = pl.ds(j * LANES, LANES)
                        rows_v[buf, r, sl] = rows_v[buf, r, sl] + pos_v[r, sl]
                    return rcarry

                lax.fori_loop(0, CHUNK, row_body, 0)
                fire_writes(c, buf)
            return carry

        lax.fori_loop(0, NCHUNK // 2, iter_body, 0)
        wait_writes(NCHUNK - 2, 0)
        wait_writes(NCHUNK - 1, 1)

    return k


def kernel(seq, table):
    idx = seq.astype(jnp.int32).reshape(TOTAL_CHUNKS, NG, GSZ)
    pos_tiled = jnp.tile(_pos_encoding(), (CHUNK // SEQ, 1))
    return _make_sc_kernel()(idx, table, pos_tiled)
